# Initial kernel scaffold; baseline (speedup 1.0000x reference)
import jax
import jax.numpy as jnp
from jax.experimental import pallas as pl


def kernel(x, edge_index, edge_attr, batch, nn1_w1, nn1_b1, nn1_w2, nn1_b2, root1, bias1, bn1_g, bn1_b, nn2_w1, nn2_b1, nn2_w2, nn2_b2, root2, bias2, bn2_g, bn2_b, lin1_w, lin1_b, lin2_w, lin2_b):
    raise NotImplementedError("write your pallas kernel here")


# speedup vs baseline: 1.7928x; 1.7928x over previous
import functools

import jax
import jax.numpy as jnp
from jax import lax
from jax.experimental import pallas as pl
from jax.experimental.pallas import tpu as pltpu
from jax.experimental.pallas import tpu_sc as plsc

N = 10000
E = 30000
G = 256
NTILES = 32
EPAD = 30720
EC = EPAD // NTILES
SCH = EC // 4
NPAD = 10240
NROWS = NPAD // 16
D1 = 80
D2 = 128
TE1 = 1920
TE2 = 960


def _sc_mesh():
    return plsc.VectorSubcoreMesh(core_axis_name="c", subcore_axis_name="s",
                                  num_cores=2, num_subcores=16)


@functools.lru_cache(maxsize=None)
def _make_gather(D):
    @functools.partial(
        pl.kernel,
        out_type=jax.ShapeDtypeStruct((EPAD, D), jnp.float32),
        mesh=_sc_mesh(),
        compiler_params=pltpu.CompilerParams(use_tc_tiling_on_sc=False),
        scratch_types=[
            pltpu.VMEM((EC,), jnp.int32),
            pltpu.VMEM((EC, D), jnp.float32),
            pltpu.SemaphoreType.DMA,
        ],
    )
    def gather_k(table_hbm, idx_hbm, out_hbm, idx_v, rows_v, sem):
        wid = lax.axis_index("s") * 2 + lax.axis_index("c")
        base = wid * EC
        pltpu.sync_copy(idx_hbm.at[pl.ds(base, EC)], idx_v)
        pltpu.async_copy(table_hbm.at[idx_v], rows_v, sem).wait()
        pltpu.sync_copy(rows_v, out_hbm.at[pl.ds(base, EC)])

    return gather_k


@functools.lru_cache(maxsize=None)
def _make_scatter(D):
    @functools.partial(
        pl.kernel,
        out_type=jax.ShapeDtypeStruct((2, NPAD, D), jnp.float32),
        mesh=_sc_mesh(),
        compiler_params=pltpu.CompilerParams(use_tc_tiling_on_sc=False),
        scratch_types=[
            pltpu.VMEM((SCH,), jnp.int32),
            pltpu.VMEM((SCH, D), jnp.float32),
            pltpu.VMEM_SHARED((NPAD, D), jnp.float32),
        ],
    )
    def scatter_k(msg_hbm, dst_hbm, zero_hbm, out_hbm, idx_v, rows_v, acc_sh):
        cid = lax.axis_index("c")
        sid = lax.axis_index("s")
        wid = cid * 16 + sid
        nbase = sid * NROWS
        pltpu.sync_copy(zero_hbm.at[pl.ds(nbase, NROWS)],
                        acc_sh.at[pl.ds(nbase, NROWS)])
        plsc.subcore_barrier()
        for j in range(EC // SCH):
            base = wid * EC + j * SCH
            pltpu.sync_copy(dst_hbm.at[pl.ds(base, SCH)], idx_v)
            pltpu.sync_copy(msg_hbm.at[pl.ds(base, SCH)], rows_v)
            pltpu.sync_copy(rows_v, acc_sh.at[idx_v], add=True)
        plsc.subcore_barrier()
        pltpu.sync_copy(acc_sh.at[pl.ds(nbase, NROWS)],
                        out_hbm.at[cid, pl.ds(nbase, NROWS)])

    return scatter_k


def _bf16x3(a, b, dims):
    ah = a.astype(jnp.bfloat16)
    al = (a - ah.astype(jnp.float32)).astype(jnp.bfloat16)
    bh = b.astype(jnp.bfloat16)
    bl = (b - bh.astype(jnp.float32)).astype(jnp.bfloat16)
    d = functools.partial(lax.dot_general, dimension_numbers=dims,
                          preferred_element_type=jnp.float32)
    return d(ah, bh) + d(ah, bl) + d(al, bh)


def _mm(a, b):
    return _bf16x3(a, b, (((1,), (0,)), ((), ())))


def _msg1_body(ea_ref, xs_ref, w1_ref, b1_ref, w2f_ref, b2m_ref, out_ref):
    i = pl.program_id(0)
    ea = ea_ref[...]
    xs = xs_ref[...]
    h = jnp.maximum(_mm(ea, w1_ref[...]) + b1_ref[...], 0.0)
    z = (h[:, :, None] * xs[:, None, :]).reshape(TE1, 32 * 32)
    msg = _mm(z, w2f_ref[...]) + _mm(xs, b2m_ref[...])
    rows = i * TE1 + lax.broadcasted_iota(jnp.int32, (TE1, 1), 0)
    valid = rows < E
    out_ref[:, :64] = jnp.where(valid, msg, 0.0)
    lane = lax.broadcasted_iota(jnp.int32, (TE1, 16), 1)
    out_ref[:, 64:] = jnp.where(valid & (lane == 0), 1.0, 0.0)


def _msg2_body(ea_ref, hs_ref, w1_ref, b1_ref, w2f_ref, b2m_ref, out_ref):
    i = pl.program_id(0)
    hs = hs_ref[...]
    h = jnp.maximum(_mm(ea_ref[...], w1_ref[...]) + b1_ref[...], 0.0)
    z = (h[:, :, None] * hs[:, None, :]).reshape(TE2, 32 * 64)
    msg = _mm(z, w2f_ref[...]) + _mm(hs, b2m_ref[...])
    rows = i * TE2 + lax.broadcasted_iota(jnp.int32, (TE2, 1), 0)
    out_ref[...] = jnp.where(rows < E, msg, 0.0)


def _node1_body(parts_ref, x_ref, root_ref, bias_ref, g_ref, b_ref,
                h1_ref, cinv_ref):
    s = parts_ref[0] + parts_ref[1]
    agg = s[:N, :64]
    cnt = s[:N, 64:65]
    cinv = 1.0 / jnp.maximum(cnt, 1.0)
    r = agg * cinv + _mm(x_ref[...], root_ref[...]) + bias_ref[...]
    r = jnp.maximum(r, 0.0)
    m = jnp.mean(r, axis=0, keepdims=True)
    v = jnp.mean((r - m) * (r - m), axis=0, keepdims=True)
    h1_ref[...] = (r - m) * lax.rsqrt(v + 1e-5) * g_ref[...] + b_ref[...]
    cinv_ref[...] = cinv


def _node2_body(parts_ref, h1_ref, cinv_ref, root_ref, bias_ref, g_ref, b_ref,
                batch_ref, l1w_ref, l1b_ref, l2w_ref, l2b_ref, out_ref):
    s = parts_ref[0] + parts_ref[1]
    h1 = h1_ref[...]
    r = s[:N, :] * cinv_ref[...] + _mm(h1, root_ref[...]) + bias_ref[...]
    r = jnp.maximum(r, 0.0)
    m = jnp.mean(r, axis=0, keepdims=True)
    v = jnp.mean((r - m) * (r - m), axis=0, keepdims=True)
    h2 = (r - m) * lax.rsqrt(v + 1e-5) * g_ref[...] + b_ref[...]
    gid = lax.broadcasted_iota(jnp.int32, (N, G), 1)
    oh = (batch_ref[...] == gid).astype(jnp.float32)
    c0 = (((0,), (0,)), ((), ()))
    pooled = _bf16x3(oh, h2, c0)
    cntg = lax.dot_general(oh.astype(jnp.bfloat16),
                           jnp.ones((N, 8), jnp.bfloat16), c0,
                           preferred_element_type=jnp.float32)[:, 0:1]
    pooled = pooled / jnp.maximum(cntg, 1.0)
    hh = jnp.maximum(_mm(pooled, l1w_ref[...]) + l1b_ref[...], 0.0)
    out_ref[...] = _mm(hh, l2w_ref[...]) + l2b_ref[...]


def kernel(x, edge_index, edge_attr, batch,
           nn1_w1, nn1_b1, nn1_w2, nn1_b2, root1, bias1, bn1_g, bn1_b,
           nn2_w1, nn2_b1, nn2_w2, nn2_b2, root2, bias2, bn2_g, bn2_b,
           lin1_w, lin1_b, lin2_w, lin2_b):
    f32 = jnp.float32
    pad = EPAD - E
    srcp = jnp.concatenate([edge_index[0], jnp.zeros((pad,), jnp.int32)])
    dstp = jnp.concatenate([edge_index[1], jnp.zeros((pad,), jnp.int32)])
    eap = jnp.pad(edge_attr, ((0, pad), (0, 0)))
    w2f1 = nn1_w2.reshape(32 * 32, 64)
    b2m1 = nn1_b2.reshape(32, 64)
    w2f2 = nn2_w2.reshape(32 * 64, 128)
    b2m2 = nn2_b2.reshape(64, 128)
    zeros1 = jnp.zeros((NPAD, D1), f32)
    zeros2 = jnp.zeros((NPAD, D2), f32)

    xs = _make_gather(32)(x, srcp)

    msg1 = pl.pallas_call(
        _msg1_body,
        grid=(EPAD // TE1,),
        in_specs=[
            pl.BlockSpec((TE1, 16), lambda i: (i, 0)),
            pl.BlockSpec((TE1, 32), lambda i: (i, 0)),
            pl.BlockSpec((16, 32), lambda i: (0, 0)),
            pl.BlockSpec((1, 32), lambda i: (0, 0)),
            pl.BlockSpec((32 * 32, 64), lambda i: (0, 0)),
            pl.BlockSpec((32, 64), lambda i: (0, 0)),
        ],
        out_specs=pl.BlockSpec((TE1, D1), lambda i: (i, 0)),
        out_shape=jax.ShapeDtypeStruct((EPAD, D1), f32),
    )(eap, xs, nn1_w1, nn1_b1.reshape(1, 32), w2f1, b2m1)

    parts1 = _make_scatter(D1)(msg1, dstp, zeros1)

    h1, cinv = pl.pallas_call(
        _node1_body,
        compiler_params=pltpu.CompilerParams(
            vmem_limit_bytes=100 * 1024 * 1024),
        out_shape=[jax.ShapeDtypeStruct((N, 64), f32),
                   jax.ShapeDtypeStruct((N, 1), f32)],
    )(parts1, x, root1, bias1.reshape(1, 64),
      bn1_g.reshape(1, 64), bn1_b.reshape(1, 64))

    hs = _make_gather(64)(h1, srcp)

    msg2 = pl.pallas_call(
        _msg2_body,
        grid=(EPAD // TE2,),
        in_specs=[
            pl.BlockSpec((TE2, 16), lambda i: (i, 0)),
            pl.BlockSpec((TE2, 64), lambda i: (i, 0)),
            pl.BlockSpec((16, 32), lambda i: (0, 0)),
            pl.BlockSpec((1, 32), lambda i: (0, 0)),
            pl.BlockSpec((32 * 64, 128), lambda i: (0, 0)),
            pl.BlockSpec((64, 128), lambda i: (0, 0)),
        ],
        out_specs=pl.BlockSpec((TE2, D2), lambda i: (i, 0)),
        out_shape=jax.ShapeDtypeStruct((EPAD, D2), f32),
    )(eap, hs, nn2_w1, nn2_b1.reshape(1, 32), w2f2, b2m2)

    parts2 = _make_scatter(D2)(msg2, dstp, zeros2)

    out = pl.pallas_call(
        _node2_body,
        compiler_params=pltpu.CompilerParams(
            vmem_limit_bytes=100 * 1024 * 1024),
        out_shape=jax.ShapeDtypeStruct((G, 1), f32),
    )(parts2, h1, cinv, root2, bias2.reshape(1, 128),
      bn2_g.reshape(1, 128), bn2_b.reshape(1, 128),
      batch.reshape(N, 1), lin1_w, lin1_b.reshape(1, 64),
      lin2_w, lin2_b.reshape(1, 1))

    return out.reshape(G)


# speedup vs baseline: 1.9767x; 1.1026x over previous
import functools

import jax
import jax.numpy as jnp
from jax import lax
from jax.experimental import pallas as pl
from jax.experimental.pallas import tpu as pltpu
from jax.experimental.pallas import tpu_sc as plsc

N = 10000
E = 30000
G = 256
NTILES = 32
EPAD = 30720
EC = EPAD // NTILES
SCH = EC // 4
NPAD = 10240
NROWS = NPAD // 16
D1 = 80
D2 = 128
TE1 = 1920
TE2 = 960


def _sc_mesh():
    return plsc.VectorSubcoreMesh(core_axis_name="c", subcore_axis_name="s",
                                  num_cores=2, num_subcores=16)


@functools.lru_cache(maxsize=None)
def _make_gather(D):
    @functools.partial(
        pl.kernel,
        out_type=jax.ShapeDtypeStruct((EPAD, D), jnp.float32),
        mesh=_sc_mesh(),
        compiler_params=pltpu.CompilerParams(use_tc_tiling_on_sc=False),
        scratch_types=[
            pltpu.VMEM((EC,), jnp.int32),
            pltpu.VMEM((EC, D), jnp.float32),
            pltpu.SemaphoreType.DMA,
        ],
    )
    def gather_k(table_hbm, idx_hbm, out_hbm, idx_v, rows_v, sem):
        wid = lax.axis_index("s") * 2 + lax.axis_index("c")
        base = wid * EC
        pltpu.sync_copy(idx_hbm.at[pl.ds(base, EC)], idx_v)
        pltpu.async_copy(table_hbm.at[idx_v], rows_v, sem).wait()
        pltpu.sync_copy(rows_v, out_hbm.at[pl.ds(base, EC)])

    return gather_k


@functools.lru_cache(maxsize=None)
def _make_scatter(D):
    @functools.partial(
        pl.kernel,
        out_type=jax.ShapeDtypeStruct((2, NPAD, D), jnp.float32),
        mesh=_sc_mesh(),
        compiler_params=pltpu.CompilerParams(use_tc_tiling_on_sc=False),
        scratch_types=[
            pltpu.VMEM((SCH,), jnp.int32),
            pltpu.VMEM((SCH, D), jnp.float32),
            pltpu.VMEM_SHARED((NPAD, D), jnp.float32),
        ],
    )
    def scatter_k(msg_hbm, dst_hbm, zero_hbm, out_hbm, idx_v, rows_v, acc_sh):
        cid = lax.axis_index("c")
        sid = lax.axis_index("s")
        wid = cid * 16 + sid
        nbase = sid * NROWS
        pltpu.sync_copy(zero_hbm.at[pl.ds(nbase, NROWS)],
                        acc_sh.at[pl.ds(nbase, NROWS)])
        plsc.subcore_barrier()
        for j in range(EC // SCH):
            base = wid * EC + j * SCH
            pltpu.sync_copy(dst_hbm.at[pl.ds(base, SCH)], idx_v)
            pltpu.sync_copy(msg_hbm.at[pl.ds(base, SCH)], rows_v)
            pltpu.sync_copy(rows_v, acc_sh.at[idx_v], add=True)
        plsc.subcore_barrier()
        pltpu.sync_copy(acc_sh.at[pl.ds(nbase, NROWS)],
                        out_hbm.at[cid, pl.ds(nbase, NROWS)])

    return scatter_k


def _bf16x3(a, b, dims):
    ah = a.astype(jnp.bfloat16)
    al = (a - ah.astype(jnp.float32)).astype(jnp.bfloat16)
    bh = b.astype(jnp.bfloat16)
    bl = (b - bh.astype(jnp.float32)).astype(jnp.bfloat16)
    d = functools.partial(lax.dot_general, dimension_numbers=dims,
                          preferred_element_type=jnp.float32)
    return d(ah, bh) + d(ah, bl) + d(al, bh)


def _mm(a, b):
    return _bf16x3(a, b, (((1,), (0,)), ((), ())))


def _split_mm(a, bh_ref, bl_ref):
    ah = a.astype(jnp.bfloat16)
    al = (a - ah.astype(jnp.float32)).astype(jnp.bfloat16)
    d = functools.partial(lax.dot_general,
                          dimension_numbers=(((1,), (0,)), ((), ())),
                          preferred_element_type=jnp.float32)
    return d(ah, bh_ref[...]) + d(ah, bl_ref[...]) + d(al, bh_ref[...])


def _msg1_body(ea_ref, xs_ref, w1_ref, b1_ref, wa_hi_ref, wa_lo_ref,
               b2m_ref, out_ref):
    i = pl.program_id(0)
    xs = xs_ref[...]
    h = jnp.maximum(_mm(ea_ref[...], w1_ref[...]) + b1_ref[...], 0.0)
    y = _split_mm(xs, wa_hi_ref, wa_lo_ref)
    msg = _mm(xs, b2m_ref[...])
    for k in range(32):
        msg = msg + h[:, k:k + 1] * y[:, k * 64:(k + 1) * 64]
    rows = i * TE1 + lax.broadcasted_iota(jnp.int32, (TE1, 1), 0)
    valid = rows < E
    out_ref[:, :64] = jnp.where(valid, msg, 0.0)
    lane = lax.broadcasted_iota(jnp.int32, (TE1, 16), 1)
    out_ref[:, 64:] = jnp.where(valid & (lane == 0), 1.0, 0.0)


def _msg2_body(ea_ref, hs_ref, w1_ref, b1_ref, wa_hi_ref, wa_lo_ref,
               b2m_ref, out_ref):
    i = pl.program_id(0)
    hs = hs_ref[...]
    h = jnp.maximum(_mm(ea_ref[...], w1_ref[...]) + b1_ref[...], 0.0)
    y = _split_mm(hs, wa_hi_ref, wa_lo_ref)
    msg = _mm(hs, b2m_ref[...])
    for k in range(32):
        msg = msg + h[:, k:k + 1] * y[:, k * 128:(k + 1) * 128]
    rows = i * TE2 + lax.broadcasted_iota(jnp.int32, (TE2, 1), 0)
    out_ref[...] = jnp.where(rows < E, msg, 0.0)


def _node1_body(parts_ref, x_ref, root_ref, bias_ref, g_ref, b_ref,
                h1_ref, cinv_ref):
    s = parts_ref[0] + parts_ref[1]
    agg = s[:N, :64]
    cnt = s[:N, 64:65]
    cinv = 1.0 / jnp.maximum(cnt, 1.0)
    r = agg * cinv + _mm(x_ref[...], root_ref[...]) + bias_ref[...]
    r = jnp.maximum(r, 0.0)
    m = jnp.mean(r, axis=0, keepdims=True)
    v = jnp.mean((r - m) * (r - m), axis=0, keepdims=True)
    h1_ref[...] = (r - m) * lax.rsqrt(v + 1e-5) * g_ref[...] + b_ref[...]
    cinv_ref[...] = cinv


def _node2_body(parts_ref, h1_ref, cinv_ref, root_ref, bias_ref, g_ref, b_ref,
                batch_ref, l1w_ref, l1b_ref, l2w_ref, l2b_ref, out_ref):
    s = parts_ref[0] + parts_ref[1]
    h1 = h1_ref[...]
    r = s[:N, :] * cinv_ref[...] + _mm(h1, root_ref[...]) + bias_ref[...]
    r = jnp.maximum(r, 0.0)
    m = jnp.mean(r, axis=0, keepdims=True)
    v = jnp.mean((r - m) * (r - m), axis=0, keepdims=True)
    h2 = (r - m) * lax.rsqrt(v + 1e-5) * g_ref[...] + b_ref[...]
    gid = lax.broadcasted_iota(jnp.int32, (N, G), 1)
    oh = (batch_ref[...] == gid).astype(jnp.float32)
    c0 = (((0,), (0,)), ((), ()))
    ohb = oh.astype(jnp.bfloat16)
    h2h = h2.astype(jnp.bfloat16)
    h2l = (h2 - h2h.astype(jnp.float32)).astype(jnp.bfloat16)
    d0 = functools.partial(lax.dot_general, dimension_numbers=c0,
                           preferred_element_type=jnp.float32)
    pooled = d0(ohb, h2h) + d0(ohb, h2l)
    cntg = lax.dot_general(oh.astype(jnp.bfloat16),
                           jnp.ones((N, 8), jnp.bfloat16), c0,
                           preferred_element_type=jnp.float32)[:, 0:1]
    pooled = pooled / jnp.maximum(cntg, 1.0)
    hh = jnp.maximum(_mm(pooled, l1w_ref[...]) + l1b_ref[...], 0.0)
    out_ref[...] = _mm(hh, l2w_ref[...]) + l2b_ref[...]


def kernel(x, edge_index, edge_attr, batch,
           nn1_w1, nn1_b1, nn1_w2, nn1_b2, root1, bias1, bn1_g, bn1_b,
           nn2_w1, nn2_b1, nn2_w2, nn2_b2, root2, bias2, bn2_g, bn2_b,
           lin1_w, lin1_b, lin2_w, lin2_b):
    f32 = jnp.float32
    pad = EPAD - E
    srcp = jnp.concatenate([edge_index[0], jnp.zeros((pad,), jnp.int32)])
    dstp = jnp.concatenate([edge_index[1], jnp.zeros((pad,), jnp.int32)])
    eap = jnp.pad(edge_attr, ((0, pad), (0, 0)))
    w1a = nn1_w2.reshape(32, 32, 64).transpose(1, 0, 2).reshape(32, 2048)
    w1a_hi = w1a.astype(jnp.bfloat16)
    w1a_lo = (w1a - w1a_hi.astype(f32)).astype(jnp.bfloat16)
    b2m1 = nn1_b2.reshape(32, 64)
    w2a = nn2_w2.reshape(32, 64, 128).transpose(1, 0, 2).reshape(64, 4096)
    w2a_hi = w2a.astype(jnp.bfloat16)
    w2a_lo = (w2a - w2a_hi.astype(f32)).astype(jnp.bfloat16)
    b2m2 = nn2_b2.reshape(64, 128)
    zeros1 = jnp.zeros((NPAD, D1), f32)
    zeros2 = jnp.zeros((NPAD, D2), f32)

    xs = _make_gather(32)(x, srcp)

    msg1 = pl.pallas_call(
        _msg1_body,
        grid=(EPAD // TE1,),
        in_specs=[
            pl.BlockSpec((TE1, 16), lambda i: (i, 0)),
            pl.BlockSpec((TE1, 32), lambda i: (i, 0)),
            pl.BlockSpec((16, 32), lambda i: (0, 0)),
            pl.BlockSpec((1, 32), lambda i: (0, 0)),
            pl.BlockSpec((32, 2048), lambda i: (0, 0)),
            pl.BlockSpec((32, 2048), lambda i: (0, 0)),
            pl.BlockSpec((32, 64), lambda i: (0, 0)),
        ],
        out_specs=pl.BlockSpec((TE1, D1), lambda i: (i, 0)),
        out_shape=jax.ShapeDtypeStruct((EPAD, D1), f32),
        compiler_params=pltpu.CompilerParams(
            vmem_limit_bytes=100 * 1024 * 1024),
    )(eap, xs, nn1_w1, nn1_b1.reshape(1, 32), w1a_hi, w1a_lo, b2m1)

    parts1 = _make_scatter(D1)(msg1, dstp, zeros1)

    h1, cinv = pl.pallas_call(
        _node1_body,
        compiler_params=pltpu.CompilerParams(
            vmem_limit_bytes=100 * 1024 * 1024),
        out_shape=[jax.ShapeDtypeStruct((N, 64), f32),
                   jax.ShapeDtypeStruct((N, 1), f32)],
    )(parts1, x, root1, bias1.reshape(1, 64),
      bn1_g.reshape(1, 64), bn1_b.reshape(1, 64))

    hs = _make_gather(64)(h1, srcp)

    msg2 = pl.pallas_call(
        _msg2_body,
        grid=(EPAD // TE2,),
        in_specs=[
            pl.BlockSpec((TE2, 16), lambda i: (i, 0)),
            pl.BlockSpec((TE2, 64), lambda i: (i, 0)),
            pl.BlockSpec((16, 32), lambda i: (0, 0)),
            pl.BlockSpec((1, 32), lambda i: (0, 0)),
            pl.BlockSpec((64, 4096), lambda i: (0, 0)),
            pl.BlockSpec((64, 4096), lambda i: (0, 0)),
            pl.BlockSpec((64, 128), lambda i: (0, 0)),
        ],
        out_specs=pl.BlockSpec((TE2, D2), lambda i: (i, 0)),
        out_shape=jax.ShapeDtypeStruct((EPAD, D2), f32),
        compiler_params=pltpu.CompilerParams(
            vmem_limit_bytes=100 * 1024 * 1024),
    )(eap, hs, nn2_w1, nn2_b1.reshape(1, 32), w2a_hi, w2a_lo, b2m2)

    parts2 = _make_scatter(D2)(msg2, dstp, zeros2)

    out = pl.pallas_call(
        _node2_body,
        compiler_params=pltpu.CompilerParams(
            vmem_limit_bytes=100 * 1024 * 1024),
        out_shape=jax.ShapeDtypeStruct((G, 1), f32),
    )(parts2, h1, cinv, root2, bias2.reshape(1, 128),
      bn2_g.reshape(1, 128), bn2_b.reshape(1, 128),
      batch.reshape(N, 1), lin1_w, lin1_b.reshape(1, 64),
      lin2_w, lin2_b.reshape(1, 1))

    return out.reshape(G)


# speedup vs baseline: 2.5744x; 1.3024x over previous
import functools

import jax
import jax.numpy as jnp
from jax import lax
from jax.experimental import pallas as pl
from jax.experimental.pallas import tpu as pltpu
from jax.experimental.pallas import tpu_sc as plsc

N = 10000
E = 30000
G = 256
NTILES = 32
EPAD = 30720
EC = EPAD // NTILES
SCH = EC // 4
NPAD = 10240
NROWS = NPAD // 16
D1 = 80
D2 = 128
TE1 = 1920
TE2 = 960


def _sc_mesh():
    return plsc.VectorSubcoreMesh(core_axis_name="c", subcore_axis_name="s",
                                  num_cores=2, num_subcores=16)


@functools.lru_cache(maxsize=None)
def _make_gather(D):
    @functools.partial(
        pl.kernel,
        out_type=jax.ShapeDtypeStruct((EPAD, D), jnp.float32),
        mesh=_sc_mesh(),
        compiler_params=pltpu.CompilerParams(use_tc_tiling_on_sc=False),
        scratch_types=[
            pltpu.VMEM((EC,), jnp.int32),
            pltpu.VMEM((EC, D), jnp.float32),
            pltpu.SemaphoreType.DMA,
        ],
    )
    def gather_k(table_hbm, idx_hbm, out_hbm, idx_v, rows_v, sem):
        wid = lax.axis_index("s") * 2 + lax.axis_index("c")
        base = wid * EC
        pltpu.sync_copy(idx_hbm.at[pl.ds(base, EC)], idx_v)
        pltpu.async_copy(table_hbm.at[idx_v], rows_v, sem).wait()
        pltpu.sync_copy(rows_v, out_hbm.at[pl.ds(base, EC)])

    return gather_k


@functools.lru_cache(maxsize=None)
def _make_scatter(D):
    @functools.partial(
        pl.kernel,
        out_type=jax.ShapeDtypeStruct((2, NPAD, D), jnp.float32),
        mesh=_sc_mesh(),
        compiler_params=pltpu.CompilerParams(use_tc_tiling_on_sc=False),
        scratch_types=[
            pltpu.VMEM((SCH,), jnp.int32),
            pltpu.VMEM((SCH, D), jnp.float32),
            pltpu.VMEM_SHARED((NPAD, D), jnp.float32),
        ],
    )
    def scatter_k(msg_hbm, dst_hbm, zero_hbm, out_hbm, idx_v, rows_v, acc_sh):
        cid = lax.axis_index("c")
        sid = lax.axis_index("s")
        wid = cid * 16 + sid
        nbase = sid * NROWS
        pltpu.sync_copy(zero_hbm.at[pl.ds(nbase, NROWS)],
                        acc_sh.at[pl.ds(nbase, NROWS)])
        plsc.subcore_barrier()
        for j in range(EC // SCH):
            base = wid * EC + j * SCH
            pltpu.sync_copy(dst_hbm.at[pl.ds(base, SCH)], idx_v)
            pltpu.sync_copy(msg_hbm.at[pl.ds(base, SCH)], rows_v)
            pltpu.sync_copy(rows_v, acc_sh.at[idx_v], add=True)
        plsc.subcore_barrier()
        pltpu.sync_copy(acc_sh.at[pl.ds(nbase, NROWS)],
                        out_hbm.at[cid, pl.ds(nbase, NROWS)])

    return scatter_k


def _bf16x3(a, b, dims):
    ah = a.astype(jnp.bfloat16)
    al = (a - ah.astype(jnp.float32)).astype(jnp.bfloat16)
    bh = b.astype(jnp.bfloat16)
    bl = (b - bh.astype(jnp.float32)).astype(jnp.bfloat16)
    d = functools.partial(lax.dot_general, dimension_numbers=dims,
                          preferred_element_type=jnp.float32)
    return d(ah, bh) + d(ah, bl) + d(al, bh)


def _mm(a, b):
    return _bf16x3(a, b, (((1,), (0,)), ((), ())))


def _split_mm(a, bh_ref, bc_ref):
    ah = a.astype(jnp.bfloat16)
    al = (a - ah.astype(jnp.float32)).astype(jnp.bfloat16)
    d = functools.partial(lax.dot_general,
                          dimension_numbers=(((1,), (0,)), ((), ())),
                          preferred_element_type=jnp.float32)
    corr = d(jnp.concatenate([ah, al], axis=1), bc_ref[...])
    return d(ah, bh_ref[...]) + corr


def _rep(h, rd_ref):
    hh = h.astype(jnp.bfloat16)
    hl = (h - hh.astype(jnp.float32)).astype(jnp.bfloat16)
    return lax.dot_general(jnp.concatenate([hh, hl], axis=1), rd_ref[...],
                           (((1,), (0,)), ((), ())),
                           preferred_element_type=jnp.float32)


def _msg1_body(ea_ref, xs_ref, w1_ref, b1_ref, wa_hi_ref, wa_lo_ref,
               rd_ref, out_ref):
    i = pl.program_id(0)
    xs = xs_ref[...]
    h = jnp.maximum(_mm(ea_ref[...], w1_ref[...]) + b1_ref[...], 0.0)
    y = _split_mm(xs, wa_hi_ref, wa_lo_ref)
    hrep = _rep(h, rd_ref)
    prod = hrep * y[:, :2048]
    acc = prod[:, :128]
    for t in range(1, 16):
        acc = acc + prod[:, t * 128:(t + 1) * 128]
    msg = acc[:, :64] + acc[:, 64:128] + y[:, 2048:2112]
    rows = i * TE1 + lax.broadcasted_iota(jnp.int32, (TE1, 1), 0)
    valid = rows < E
    out_ref[:, :64] = jnp.where(valid, msg, 0.0)
    lane = lax.broadcasted_iota(jnp.int32, (TE1, 16), 1)
    out_ref[:, 64:] = jnp.where(valid & (lane == 0), 1.0, 0.0)


def _msg2_body(ea_ref, hs_ref, w1_ref, b1_ref, wa_hi_ref, wa_lo_ref,
               rd_ref, out_ref):
    i = pl.program_id(0)
    hs = hs_ref[...]
    h = jnp.maximum(_mm(ea_ref[...], w1_ref[...]) + b1_ref[...], 0.0)
    y = _split_mm(hs, wa_hi_ref, wa_lo_ref)
    hrep = _rep(h, rd_ref)
    prod = hrep * y[:, :4096]
    acc = prod[:, :128]
    for t in range(1, 32):
        acc = acc + prod[:, t * 128:(t + 1) * 128]
    msg = acc + y[:, 4096:4224]
    rows = i * TE2 + lax.broadcasted_iota(jnp.int32, (TE2, 1), 0)
    out_ref[...] = jnp.where(rows < E, msg, 0.0)


def _node1_body(parts_ref, x_ref, root_ref, bias_ref, g_ref, b_ref,
                h1_ref, cinv_ref):
    s = parts_ref[0] + parts_ref[1]
    agg = s[:N, :64]
    cnt = s[:N, 64:65]
    cinv = 1.0 / jnp.maximum(cnt, 1.0)
    r = agg * cinv + _mm(x_ref[...], root_ref[...]) + bias_ref[...]
    r = jnp.maximum(r, 0.0)
    m = jnp.mean(r, axis=0, keepdims=True)
    v = jnp.mean((r - m) * (r - m), axis=0, keepdims=True)
    h1_ref[...] = (r - m) * lax.rsqrt(v + 1e-5) * g_ref[...] + b_ref[...]
    cinv_ref[...] = cinv


def _node2_body(parts_ref, h1_ref, cinv_ref, root_ref, bias_ref, g_ref, b_ref,
                batch_ref, l1w_ref, l1b_ref, l2w_ref, l2b_ref, out_ref):
    s = parts_ref[0] + parts_ref[1]
    h1 = h1_ref[...]
    r = s[:N, :] * cinv_ref[...] + _mm(h1, root_ref[...]) + bias_ref[...]
    r = jnp.maximum(r, 0.0)
    m = jnp.mean(r, axis=0, keepdims=True)
    v = jnp.mean((r - m) * (r - m), axis=0, keepdims=True)
    h2 = (r - m) * lax.rsqrt(v + 1e-5) * g_ref[...] + b_ref[...]
    gid = lax.broadcasted_iota(jnp.int32, (N, G), 1)
    oh = (batch_ref[...] == gid).astype(jnp.float32)
    c0 = (((0,), (0,)), ((), ()))
    ohb = oh.astype(jnp.bfloat16)
    h2h = h2.astype(jnp.bfloat16)
    h2l = (h2 - h2h.astype(jnp.float32)).astype(jnp.bfloat16)
    d0 = functools.partial(lax.dot_general, dimension_numbers=c0,
                           preferred_element_type=jnp.float32)
    pooled = d0(ohb, h2h) + d0(ohb, h2l)
    cntg = lax.dot_general(oh.astype(jnp.bfloat16),
                           jnp.ones((N, 8), jnp.bfloat16), c0,
                           preferred_element_type=jnp.float32)[:, 0:1]
    pooled = pooled / jnp.maximum(cntg, 1.0)
    hh = jnp.maximum(_mm(pooled, l1w_ref[...]) + l1b_ref[...], 0.0)
    out_ref[...] = _mm(hh, l2w_ref[...]) + l2b_ref[...]


def kernel(x, edge_index, edge_attr, batch,
           nn1_w1, nn1_b1, nn1_w2, nn1_b2, root1, bias1, bn1_g, bn1_b,
           nn2_w1, nn2_b1, nn2_w2, nn2_b2, root2, bias2, bn2_g, bn2_b,
           lin1_w, lin1_b, lin2_w, lin2_b):
    f32 = jnp.float32
    pad = EPAD - E
    srcp = jnp.concatenate([edge_index[0], jnp.zeros((pad,), jnp.int32)])
    dstp = jnp.concatenate([edge_index[1], jnp.zeros((pad,), jnp.int32)])
    eap = jnp.pad(edge_attr, ((0, pad), (0, 0)))
    w1a = nn1_w2.reshape(32, 32, 64).transpose(1, 0, 2).reshape(32, 2048)
    w1a = jnp.concatenate([w1a, nn1_b2.reshape(32, 64)], axis=1)
    w1a_hi = w1a.astype(jnp.bfloat16)
    w1a_lo = (w1a - w1a_hi.astype(f32)).astype(jnp.bfloat16)
    w1a_c = jnp.concatenate([w1a_lo, w1a_hi], axis=0)
    r1 = (jnp.arange(2048)[None, :] // 64 == jnp.arange(32)[:, None])
    r1d = jnp.concatenate([r1, r1], axis=0).astype(jnp.bfloat16)
    w2a = nn2_w2.reshape(32, 64, 128).transpose(1, 0, 2).reshape(64, 4096)
    w2a = jnp.concatenate([w2a, nn2_b2.reshape(64, 128)], axis=1)
    w2a_hi = w2a.astype(jnp.bfloat16)
    w2a_lo = (w2a - w2a_hi.astype(f32)).astype(jnp.bfloat16)
    w2a_c = jnp.concatenate([w2a_lo, w2a_hi], axis=0)
    r2 = (jnp.arange(4096)[None, :] // 128 == jnp.arange(32)[:, None])
    r2d = jnp.concatenate([r2, r2], axis=0).astype(jnp.bfloat16)
    zeros1 = jnp.zeros((NPAD, D1), f32)
    zeros2 = jnp.zeros((NPAD, D2), f32)

    xs = _make_gather(32)(x, srcp)

    msg1 = pl.pallas_call(
        _msg1_body,
        grid=(EPAD // TE1,),
        in_specs=[
            pl.BlockSpec((TE1, 16), lambda i: (i, 0)),
            pl.BlockSpec((TE1, 32), lambda i: (i, 0)),
            pl.BlockSpec((16, 32), lambda i: (0, 0)),
            pl.BlockSpec((1, 32), lambda i: (0, 0)),
            pl.BlockSpec((32, 2112), lambda i: (0, 0)),
            pl.BlockSpec((64, 2112), lambda i: (0, 0)),
            pl.BlockSpec((64, 2048), lambda i: (0, 0)),
        ],
        out_specs=pl.BlockSpec((TE1, D1), lambda i: (i, 0)),
        out_shape=jax.ShapeDtypeStruct((EPAD, D1), f32),
        compiler_params=pltpu.CompilerParams(
            vmem_limit_bytes=100 * 1024 * 1024),
    )(eap, xs, nn1_w1, nn1_b1.reshape(1, 32), w1a_hi, w1a_c, r1d)

    parts1 = _make_scatter(D1)(msg1, dstp, zeros1)

    h1, cinv = pl.pallas_call(
        _node1_body,
        compiler_params=pltpu.CompilerParams(
            vmem_limit_bytes=100 * 1024 * 1024),
        out_shape=[jax.ShapeDtypeStruct((N, 64), f32),
                   jax.ShapeDtypeStruct((N, 1), f32)],
    )(parts1, x, root1, bias1.reshape(1, 64),
      bn1_g.reshape(1, 64), bn1_b.reshape(1, 64))

    hs = _make_gather(64)(h1, srcp)

    msg2 = pl.pallas_call(
        _msg2_body,
        grid=(EPAD // TE2,),
        in_specs=[
            pl.BlockSpec((TE2, 16), lambda i: (i, 0)),
            pl.BlockSpec((TE2, 64), lambda i: (i, 0)),
            pl.BlockSpec((16, 32), lambda i: (0, 0)),
            pl.BlockSpec((1, 32), lambda i: (0, 0)),
            pl.BlockSpec((64, 4224), lambda i: (0, 0)),
            pl.BlockSpec((128, 4224), lambda i: (0, 0)),
            pl.BlockSpec((64, 4096), lambda i: (0, 0)),
        ],
        out_specs=pl.BlockSpec((TE2, D2), lambda i: (i, 0)),
        out_shape=jax.ShapeDtypeStruct((EPAD, D2), f32),
        compiler_params=pltpu.CompilerParams(
            vmem_limit_bytes=100 * 1024 * 1024),
    )(eap, hs, nn2_w1, nn2_b1.reshape(1, 32), w2a_hi, w2a_c, r2d)

    parts2 = _make_scatter(D2)(msg2, dstp, zeros2)

    out = pl.pallas_call(
        _node2_body,
        compiler_params=pltpu.CompilerParams(
            vmem_limit_bytes=100 * 1024 * 1024),
        out_shape=jax.ShapeDtypeStruct((G, 1), f32),
    )(parts2, h1, cinv, root2, bias2.reshape(1, 128),
      bn2_g.reshape(1, 128), bn2_b.reshape(1, 128),
      batch.reshape(N, 1), lin1_w, lin1_b.reshape(1, 64),
      lin2_w, lin2_b.reshape(1, 1))

    return out.reshape(G)


# speedup vs baseline: 2.6198x; 1.0176x over previous
import functools

import jax
import jax.numpy as jnp
from jax import lax
from jax.experimental import pallas as pl
from jax.experimental.pallas import tpu as pltpu
from jax.experimental.pallas import tpu_sc as plsc

N = 10000
E = 30000
G = 256
NTILES = 32
EPAD = 30720
EC = EPAD // NTILES
NCH = 8
SCH = EC // NCH
GCH = EC // 2
NPAD = 10240
NROWS = NPAD // 16
D1 = 80
D2 = 128
TE1 = 1920
TE2 = 960


def _sc_mesh():
    return plsc.VectorSubcoreMesh(core_axis_name="c", subcore_axis_name="s",
                                  num_cores=2, num_subcores=16)


@functools.lru_cache(maxsize=None)
def _make_gather(D):
    @functools.partial(
        pl.kernel,
        out_type=jax.ShapeDtypeStruct((EPAD, D), jnp.float32),
        mesh=_sc_mesh(),
        compiler_params=pltpu.CompilerParams(use_tc_tiling_on_sc=False),
        scratch_types=[
            pltpu.VMEM((2, GCH), jnp.int32),
            pltpu.VMEM((2, GCH, D), jnp.float32),
            pltpu.SemaphoreType.DMA,
            pltpu.SemaphoreType.DMA,
        ],
    )
    def gather_k(table_hbm, idx_hbm, out_hbm, idx_v, rows_v, sem0, sem1):
        wid = lax.axis_index("s") * 2 + lax.axis_index("c")
        base = wid * EC
        pltpu.sync_copy(idx_hbm.at[wid], idx_v)
        d0 = pltpu.async_copy(table_hbm.at[idx_v.at[0]], rows_v.at[0], sem0)
        d1 = pltpu.async_copy(table_hbm.at[idx_v.at[1]], rows_v.at[1], sem1)
        d0.wait()
        pltpu.sync_copy(rows_v.at[0], out_hbm.at[pl.ds(base, GCH)])
        d1.wait()
        pltpu.sync_copy(rows_v.at[1], out_hbm.at[pl.ds(base + GCH, GCH)])

    return gather_k


@functools.lru_cache(maxsize=None)
def _make_scatter(D):
    @functools.partial(
        pl.kernel,
        out_type=jax.ShapeDtypeStruct((2, NPAD, D), jnp.float32),
        mesh=_sc_mesh(),
        compiler_params=pltpu.CompilerParams(use_tc_tiling_on_sc=False),
        scratch_types=[
            pltpu.VMEM((NCH, SCH), jnp.int32),
            pltpu.VMEM((2, SCH, D), jnp.float32),
            pltpu.VMEM_SHARED((NPAD, D), jnp.float32),
            pltpu.SemaphoreType.DMA,
            pltpu.SemaphoreType.DMA,
        ],
    )
    def scatter_k(msg_hbm, dst_hbm, zero_hbm, out_hbm, idx_v, rows_v, acc_sh,
                  sem0, sem1):
        cid = lax.axis_index("c")
        sid = lax.axis_index("s")
        wid = cid * 16 + sid
        nbase = sid * NROWS
        pltpu.sync_copy(zero_hbm.at[pl.ds(nbase, NROWS)],
                        acc_sh.at[pl.ds(nbase, NROWS)])
        pltpu.sync_copy(dst_hbm.at[wid], idx_v)
        sems = (sem0, sem1)
        base = wid * EC
        pend = pltpu.async_copy(msg_hbm.at[pl.ds(base, SCH)],
                                rows_v.at[0], sems[0])
        plsc.subcore_barrier()
        for j in range(NCH):
            nxt = None
            if j + 1 < NCH:
                nxt = pltpu.async_copy(
                    msg_hbm.at[pl.ds(base + (j + 1) * SCH, SCH)],
                    rows_v.at[(j + 1) % 2], sems[(j + 1) % 2])
            pend.wait()
            pltpu.sync_copy(rows_v.at[j % 2], acc_sh.at[idx_v.at[j]],
                            add=True)
            pend = nxt
        plsc.subcore_barrier()
        pltpu.sync_copy(acc_sh.at[pl.ds(nbase, NROWS)],
                        out_hbm.at[cid, pl.ds(nbase, NROWS)])

    return scatter_k


def _bf16x3(a, b, dims):
    ah = a.astype(jnp.bfloat16)
    al = (a - ah.astype(jnp.float32)).astype(jnp.bfloat16)
    bh = b.astype(jnp.bfloat16)
    bl = (b - bh.astype(jnp.float32)).astype(jnp.bfloat16)
    d = functools.partial(lax.dot_general, dimension_numbers=dims,
                          preferred_element_type=jnp.float32)
    return d(ah, bh) + d(ah, bl) + d(al, bh)


def _mm(a, b):
    return _bf16x3(a, b, (((1,), (0,)), ((), ())))


def _split_mm(a, bh_ref, bc_ref):
    ah = a.astype(jnp.bfloat16)
    al = (a - ah.astype(jnp.float32)).astype(jnp.bfloat16)
    d = functools.partial(lax.dot_general,
                          dimension_numbers=(((1,), (0,)), ((), ())),
                          preferred_element_type=jnp.float32)
    corr = d(jnp.concatenate([ah, al], axis=1), bc_ref[...])
    return d(ah, bh_ref[...]) + corr


def _rep(h, rd_ref):
    hh = h.astype(jnp.bfloat16)
    hl = (h - hh.astype(jnp.float32)).astype(jnp.bfloat16)
    return lax.dot_general(jnp.concatenate([hh, hl], axis=1), rd_ref[...],
                           (((1,), (0,)), ((), ())),
                           preferred_element_type=jnp.float32)


def _msg1_body(ea_ref, xs_ref, w1_ref, b1_ref, wa_hi_ref, wa_lo_ref,
               rd_ref, out_ref):
    i = pl.program_id(0)
    xs = xs_ref[...]
    h = jnp.maximum(_mm(ea_ref[...], w1_ref[...]) + b1_ref[...], 0.0)
    y = _split_mm(xs, wa_hi_ref, wa_lo_ref)
    hrep = _rep(h, rd_ref)
    prod = hrep * y[:, :2048]
    acc = prod[:, :128]
    for t in range(1, 16):
        acc = acc + prod[:, t * 128:(t + 1) * 128]
    msg = acc[:, :64] + acc[:, 64:128] + y[:, 2048:2112]
    rows = i * TE1 + lax.broadcasted_iota(jnp.int32, (TE1, 1), 0)
    valid = rows < E
    out_ref[:, :64] = jnp.where(valid, msg, 0.0)
    lane = lax.broadcasted_iota(jnp.int32, (TE1, 16), 1)
    out_ref[:, 64:] = jnp.where(valid & (lane == 0), 1.0, 0.0)


def _msg2_body(ea_ref, hs_ref, w1_ref, b1_ref, wa_hi_ref, wa_lo_ref,
               rd_ref, out_ref):
    i = pl.program_id(0)
    hs = hs_ref[...]
    h = jnp.maximum(_mm(ea_ref[...], w1_ref[...]) + b1_ref[...], 0.0)
    y = _split_mm(hs, wa_hi_ref, wa_lo_ref)
    hrep = _rep(h, rd_ref)
    prod = hrep * y[:, :4096]
    acc = prod[:, :128]
    for t in range(1, 32):
        acc = acc + prod[:, t * 128:(t + 1) * 128]
    msg = acc + y[:, 4096:4224]
    rows = i * TE2 + lax.broadcasted_iota(jnp.int32, (TE2, 1), 0)
    out_ref[...] = jnp.where(rows < E, msg, 0.0)


def _node1_body(parts_ref, x_ref, root_ref, bias_ref, g_ref, b_ref,
                h1_ref, cinv_ref):
    s = parts_ref[0] + parts_ref[1]
    agg = s[:N, :64]
    cnt = s[:N, 64:65]
    cinv = 1.0 / jnp.maximum(cnt, 1.0)
    r = agg * cinv + _mm(x_ref[...], root_ref[...]) + bias_ref[...]
    r = jnp.maximum(r, 0.0)
    m = jnp.mean(r, axis=0, keepdims=True)
    v = jnp.mean((r - m) * (r - m), axis=0, keepdims=True)
    h1_ref[...] = (r - m) * lax.rsqrt(v + 1e-5) * g_ref[...] + b_ref[...]
    cinv_ref[...] = cinv


def _node2_body(parts_ref, h1_ref, cinv_ref, root_ref, bias_ref, g_ref, b_ref,
                batch_ref, l1w_ref, l1b_ref, l2w_ref, l2b_ref, out_ref):
    s = parts_ref[0] + parts_ref[1]
    h1 = h1_ref[...]
    r = s[:N, :] * cinv_ref[...] + _mm(h1, root_ref[...]) + bias_ref[...]
    r = jnp.maximum(r, 0.0)
    m = jnp.mean(r, axis=0, keepdims=True)
    v = jnp.mean((r - m) * (r - m), axis=0, keepdims=True)
    h2 = (r - m) * lax.rsqrt(v + 1e-5) * g_ref[...] + b_ref[...]
    gid = lax.broadcasted_iota(jnp.int32, (N, G), 1)
    oh = (batch_ref[...] == gid).astype(jnp.float32)
    c0 = (((0,), (0,)), ((), ()))
    ohb = oh.astype(jnp.bfloat16)
    h2h = h2.astype(jnp.bfloat16)
    h2l = (h2 - h2h.astype(jnp.float32)).astype(jnp.bfloat16)
    d0 = functools.partial(lax.dot_general, dimension_numbers=c0,
                           preferred_element_type=jnp.float32)
    pooled = d0(ohb, h2h) + d0(ohb, h2l)
    cntg = lax.dot_general(oh.astype(jnp.bfloat16),
                           jnp.ones((N, 8), jnp.bfloat16), c0,
                           preferred_element_type=jnp.float32)[:, 0:1]
    pooled = pooled / jnp.maximum(cntg, 1.0)
    hh = jnp.maximum(_mm(pooled, l1w_ref[...]) + l1b_ref[...], 0.0)
    out_ref[...] = _mm(hh, l2w_ref[...]) + l2b_ref[...]


def kernel(x, edge_index, edge_attr, batch,
           nn1_w1, nn1_b1, nn1_w2, nn1_b2, root1, bias1, bn1_g, bn1_b,
           nn2_w1, nn2_b1, nn2_w2, nn2_b2, root2, bias2, bn2_g, bn2_b,
           lin1_w, lin1_b, lin2_w, lin2_b):
    f32 = jnp.float32
    pad = EPAD - E
    srcp = jnp.concatenate([edge_index[0], jnp.zeros((pad,), jnp.int32)])
    dstp = jnp.concatenate([edge_index[1], jnp.zeros((pad,), jnp.int32)])
    src3 = srcp.reshape(NTILES, 2, GCH)
    dst3 = dstp.reshape(NTILES, NCH, SCH)
    eap = jnp.pad(edge_attr, ((0, pad), (0, 0)))
    w1a = nn1_w2.reshape(32, 32, 64).transpose(1, 0, 2).reshape(32, 2048)
    w1a = jnp.concatenate([w1a, nn1_b2.reshape(32, 64)], axis=1)
    w1a_hi = w1a.astype(jnp.bfloat16)
    w1a_lo = (w1a - w1a_hi.astype(f32)).astype(jnp.bfloat16)
    w1a_c = jnp.concatenate([w1a_lo, w1a_hi], axis=0)
    r1 = (jnp.arange(2048)[None, :] // 64 == jnp.arange(32)[:, None])
    r1d = jnp.concatenate([r1, r1], axis=0).astype(jnp.bfloat16)
    w2a = nn2_w2.reshape(32, 64, 128).transpose(1, 0, 2).reshape(64, 4096)
    w2a = jnp.concatenate([w2a, nn2_b2.reshape(64, 128)], axis=1)
    w2a_hi = w2a.astype(jnp.bfloat16)
    w2a_lo = (w2a - w2a_hi.astype(f32)).astype(jnp.bfloat16)
    w2a_c = jnp.concatenate([w2a_lo, w2a_hi], axis=0)
    r2 = (jnp.arange(4096)[None, :] // 128 == jnp.arange(32)[:, None])
    r2d = jnp.concatenate([r2, r2], axis=0).astype(jnp.bfloat16)
    zeros1 = jnp.zeros((NPAD, D1), f32)
    zeros2 = jnp.zeros((NPAD, D2), f32)

    xs = _make_gather(32)(x, src3)

    msg1 = pl.pallas_call(
        _msg1_body,
        grid=(EPAD // TE1,),
        in_specs=[
            pl.BlockSpec((TE1, 16), lambda i: (i, 0)),
            pl.BlockSpec((TE1, 32), lambda i: (i, 0)),
            pl.BlockSpec((16, 32), lambda i: (0, 0)),
            pl.BlockSpec((1, 32), lambda i: (0, 0)),
            pl.BlockSpec((32, 2112), lambda i: (0, 0)),
            pl.BlockSpec((64, 2112), lambda i: (0, 0)),
            pl.BlockSpec((64, 2048), lambda i: (0, 0)),
        ],
        out_specs=pl.BlockSpec((TE1, D1), lambda i: (i, 0)),
        out_shape=jax.ShapeDtypeStruct((EPAD, D1), f32),
        compiler_params=pltpu.CompilerParams(
            vmem_limit_bytes=100 * 1024 * 1024),
    )(eap, xs, nn1_w1, nn1_b1.reshape(1, 32), w1a_hi, w1a_c, r1d)

    parts1 = _make_scatter(D1)(msg1, dst3, zeros1)

    h1, cinv = pl.pallas_call(
        _node1_body,
        compiler_params=pltpu.CompilerParams(
            vmem_limit_bytes=100 * 1024 * 1024),
        out_shape=[jax.ShapeDtypeStruct((N, 64), f32),
                   jax.ShapeDtypeStruct((N, 1), f32)],
    )(parts1, x, root1, bias1.reshape(1, 64),
      bn1_g.reshape(1, 64), bn1_b.reshape(1, 64))

    hs = _make_gather(64)(h1, src3)

    msg2 = pl.pallas_call(
        _msg2_body,
        grid=(EPAD // TE2,),
        in_specs=[
            pl.BlockSpec((TE2, 16), lambda i: (i, 0)),
            pl.BlockSpec((TE2, 64), lambda i: (i, 0)),
            pl.BlockSpec((16, 32), lambda i: (0, 0)),
            pl.BlockSpec((1, 32), lambda i: (0, 0)),
            pl.BlockSpec((64, 4224), lambda i: (0, 0)),
            pl.BlockSpec((128, 4224), lambda i: (0, 0)),
            pl.BlockSpec((64, 4096), lambda i: (0, 0)),
        ],
        out_specs=pl.BlockSpec((TE2, D2), lambda i: (i, 0)),
        out_shape=jax.ShapeDtypeStruct((EPAD, D2), f32),
        compiler_params=pltpu.CompilerParams(
            vmem_limit_bytes=100 * 1024 * 1024),
    )(eap, hs, nn2_w1, nn2_b1.reshape(1, 32), w2a_hi, w2a_c, r2d)

    parts2 = _make_scatter(D2)(msg2, dst3, zeros2)

    out = pl.pallas_call(
        _node2_body,
        compiler_params=pltpu.CompilerParams(
            vmem_limit_bytes=100 * 1024 * 1024),
        out_shape=jax.ShapeDtypeStruct((G, 1), f32),
    )(parts2, h1, cinv, root2, bias2.reshape(1, 128),
      bn2_g.reshape(1, 128), bn2_b.reshape(1, 128),
      batch.reshape(N, 1), lin1_w, lin1_b.reshape(1, 64),
      lin2_w, lin2_b.reshape(1, 1))

    return out.reshape(G)
